# Initial kernel scaffold; baseline (speedup 1.0000x reference)
import jax
import jax.numpy as jnp
from jax.experimental import pallas as pl


def kernel(f, knn_idx, W1, b1, W2, b2, W3, b3):
    raise NotImplementedError("write your pallas kernel here")


# speedup vs baseline: 5.1361x; 5.1361x over previous
import functools

import jax
import jax.numpy as jnp
from jax import lax
from jax.experimental import pallas as pl
from jax.experimental.pallas import tpu as pltpu
from jax.experimental.pallas import tpu_sc as plsc


def _stage1_body(f_ref, w1_ref, out_ref):
    c = f_ref.shape[1]
    w1 = w1_ref[...]
    wb = w1[:, c:2 * c] + w1[:, 2 * c:]
    out_ref[...] = lax.dot_general(
        f_ref[...], wb, (((1,), (1,)), ((), ())),
        preferred_element_type=jnp.float32)


@functools.cache
def _make_gather(nk, h, chunk):
    info = plsc.get_sparse_core_info()
    nw = info.num_cores * info.num_subcores
    per_w = nk // nw
    nch = per_w // chunk
    mesh = plsc.VectorSubcoreMesh(core_axis_name="c", subcore_axis_name="s")

    def body(table_hbm, idx_hbm, out_hbm, idx_v, rows_v, sem):
        wid = lax.axis_index("s") * info.num_cores + lax.axis_index("c")
        base = wid * per_w
        for c in range(nch):
            off = base + c * chunk
            pltpu.sync_copy(idx_hbm.at[pl.ds(off, chunk)], idx_v)
            pltpu.async_copy(table_hbm.at[idx_v], rows_v, sem).wait()
            pltpu.sync_copy(rows_v, out_hbm.at[pl.ds(off, chunk)])

    return pl.kernel(
        body,
        out_type=jax.ShapeDtypeStruct((nk, h), jnp.float32),
        mesh=mesh,
        scratch_types=[
            pltpu.VMEM((chunk,), jnp.int32),
            pltpu.VMEM((chunk, h), jnp.float32),
            pltpu.SemaphoreType.DMA,
        ],
    )


def _stage3_body(f_ref, g_ref, w1_ref, b1_ref, w2_ref, b2_ref, w3_ref,
                 b3_ref, out_ref):
    bn, c = f_ref.shape
    k = g_ref.shape[0] // bn
    h = w2_ref.shape[0]
    w1 = w1_ref[...]
    wa = w1[:, :c] - w1[:, 2 * c:]
    a = lax.dot_general(f_ref[...], wa, (((1,), (1,)), ((), ())),
                        preferred_element_type=jnp.float32) + b1_ref[...]
    g = g_ref[...].reshape(bn, k, h)
    h1 = jnp.maximum(g + a[:, None, :], 0.0).reshape(bn * k, h)
    h2 = lax.dot_general(h1, w2_ref[...], (((1,), (1,)), ((), ())),
                         preferred_element_type=jnp.float32) + b2_ref[...]
    h2 = jnp.maximum(h2, 0.0)
    pooled = jnp.max(h2.reshape(bn, k, h), axis=1)
    out_ref[...] = lax.dot_general(
        pooled, w3_ref[...], (((1,), (1,)), ((), ())),
        preferred_element_type=jnp.float32) + b3_ref[...]


def kernel(f, knn_idx, W1, b1, W2, b2, W3, b3):
    B, N, C = f.shape
    K = knn_idx.shape[-1]
    H = W1.shape[0]
    O = W3.shape[0]
    NK = N * K
    BN = 1000

    f2 = f.reshape(N, C)
    idx = knn_idx.reshape(NK).astype(jnp.int32)

    bnb = pl.pallas_call(
        _stage1_body,
        out_shape=jax.ShapeDtypeStruct((N, H), jnp.float32),
    )(f2, W1)

    g = _make_gather(NK, H, 200)(bnb, idx)

    grid = (N // BN,)
    out = pl.pallas_call(
        _stage3_body,
        grid=grid,
        in_specs=[
            pl.BlockSpec((BN, C), lambda i: (i, 0)),
            pl.BlockSpec((BN * K, H), lambda i: (i, 0)),
            pl.BlockSpec((H, 3 * C), lambda i: (0, 0)),
            pl.BlockSpec((1, H), lambda i: (0, 0)),
            pl.BlockSpec((H, H), lambda i: (0, 0)),
            pl.BlockSpec((1, H), lambda i: (0, 0)),
            pl.BlockSpec((O, H), lambda i: (0, 0)),
            pl.BlockSpec((1, O), lambda i: (0, 0)),
        ],
        out_specs=pl.BlockSpec((BN, O), lambda i: (i, 0)),
        out_shape=jax.ShapeDtypeStruct((N, O), jnp.float32),
    )(f2, g, W1, b1.reshape(1, H), W2, b2.reshape(1, H), W3,
      b3.reshape(1, O))

    return out.reshape(B, N, O)
